# baseline (speedup 1.0000x reference)
import functools

import jax
import jax.numpy as jnp
from jax import lax
from jax.experimental import pallas as pl
from jax.experimental.pallas import tpu as pltpu
from jax.experimental.pallas import tpu_sc as plsc

DIM = 768
HIDDEN = 2048
E = 8
N = 2048
TILE = 128
MAX_TILES = N // TILE + E - 1
PAD_N = MAX_TILES * TILE
LANES = 128
WINDOW = 64


def _shift_rows(c, k):
    return jnp.concatenate([jnp.zeros((k, c.shape[1]), c.dtype), c[:-k, :]], axis=0)


def _shift_lanes(c, k):
    return jnp.concatenate([jnp.zeros((c.shape[0], k), c.dtype), c[:, :-k]], axis=1)


def _router_body(x_ref, gw_ref, dest_ref, start_ref, tiles_ref):
    x = x_ref[...]
    gw = gw_ref[...]
    logits = lax.dot_general(x, gw, (((1,), (1,)), ((), ())),
                             preferred_element_type=jnp.float32)
    col = lax.broadcasted_iota(jnp.int32, (N, LANES), 1)
    valid = col < E
    lm = jnp.where(valid, logits, -jnp.inf)
    m = jnp.max(lm, axis=1, keepdims=True)
    ex = jnp.exp(lm - m)
    p = ex / jnp.sum(ex, axis=1, keepdims=True)
    pmax = jnp.max(p, axis=1, keepdims=True)
    cand = jnp.where((p == pmax) & valid, col, LANES)
    eid = jnp.min(cand, axis=1, keepdims=True)
    onehot = (col == eid).astype(jnp.int32)

    c = onehot
    k = 1
    while k < N:
        c = c + _shift_rows(c, k)
        k *= 2
    counts = c[N - 1:N, :]
    rank = jnp.sum(c * onehot, axis=1, keepdims=True) - 1

    tiles = (counts + (TILE - 1)) // TILE
    cuminc = tiles
    k = 1
    while k < E:
        cuminc = cuminc + _shift_lanes(cuminc, k)
        k *= 2
    start = cuminc - tiles
    base = jnp.sum(onehot * (start * TILE), axis=1, keepdims=True)
    dest_ref[...] = base + rank
    start_ref[...] = start
    tiles_ref[...] = tiles

def _router(x_flat, gw_pad):
    return pl.pallas_call(
        _router_body,
        out_shape=(jax.ShapeDtypeStruct((N, 1), jnp.int32),
                   jax.ShapeDtypeStruct((1, LANES), jnp.int32),
                   jax.ShapeDtypeStruct((1, LANES), jnp.int32)),
    )(x_flat, gw_pad)


def _ffn_body(start_ref, tiles_ref, x_hbm, w1_ref, w3_ref, w2_ref, y_hbm,
              xt_ref, yt_ref, in_sem, out_sem):
    e = pl.program_id(0)
    start = start_ref[e]
    ntiles = tiles_ref[e]
    w1b = w1_ref[0].astype(jnp.bfloat16)
    w3b = w3_ref[0].astype(jnp.bfloat16)
    w2b = w2_ref[0].astype(jnp.bfloat16)

    def tile_step(tl, carry):
        row0 = (start + tl) * TILE
        pltpu.make_async_copy(x_hbm.at[pl.ds(row0, TILE)], xt_ref, in_sem).start()
        pltpu.make_async_copy(x_hbm.at[pl.ds(row0, TILE)], xt_ref, in_sem).wait()
        x = xt_ref[...].astype(jnp.bfloat16)
        a = lax.dot_general(x, w1b, (((1,), (1,)), ((), ())),
                            preferred_element_type=jnp.float32)
        b = lax.dot_general(x, w3b, (((1,), (1,)), ((), ())),
                            preferred_element_type=jnp.float32)
        h = (a * jax.nn.sigmoid(a) * b).astype(jnp.bfloat16)
        yt_ref[...] = lax.dot_general(h, w2b, (((1,), (1,)), ((), ())),
                                      preferred_element_type=jnp.float32)
        pltpu.make_async_copy(yt_ref, y_hbm.at[pl.ds(row0, TILE)], out_sem).start()
        pltpu.make_async_copy(yt_ref, y_hbm.at[pl.ds(row0, TILE)], out_sem).wait()
        return carry

    lax.fori_loop(0, ntiles, tile_step, 0)


def _ffn(x_sorted, w1, w3, w2, start_t, tiles_t):
    grid_spec = pltpu.PrefetchScalarGridSpec(
        num_scalar_prefetch=2,
        grid=(E,),
        in_specs=[
            pl.BlockSpec(memory_space=pl.ANY),
            pl.BlockSpec((1, HIDDEN, DIM), lambda e, s, t: (e, 0, 0)),
            pl.BlockSpec((1, HIDDEN, DIM), lambda e, s, t: (e, 0, 0)),
            pl.BlockSpec((1, DIM, HIDDEN), lambda e, s, t: (e, 0, 0)),
        ],
        out_specs=pl.BlockSpec(memory_space=pl.ANY),
        scratch_shapes=[
            pltpu.VMEM((TILE, DIM), jnp.float32),
            pltpu.VMEM((TILE, DIM), jnp.float32),
            pltpu.SemaphoreType.DMA,
            pltpu.SemaphoreType.DMA,
        ],
    )
    return pl.pallas_call(
        _ffn_body,
        grid_spec=grid_spec,
        out_shape=jax.ShapeDtypeStruct((PAD_N, DIM), jnp.float32),
    )(start_t, tiles_t, x_sorted, w1, w3, w2)


@functools.cache
def _vmesh():
    return plsc.VectorSubcoreMesh(core_axis_name="core", subcore_axis_name="subcore")


@functools.cache
def _num_cores():
    return plsc.get_sparse_core_info().num_cores


def _dispatch(x_flat, dest_1d):
    nc = _num_cores()

    @functools.partial(
        pl.kernel,
        out_type=jax.ShapeDtypeStruct((PAD_N, DIM), jnp.float32),
        mesh=_vmesh(),
        scratch_types=[pltpu.VMEM((WINDOW,), jnp.int32),
                       pltpu.VMEM((WINDOW, DIM), jnp.float32),
                       pltpu.SemaphoreType.DMA],
    )
    def k(x_hbm, i_hbm, o_hbm, idx_v, rows_v, sem):
        wid = lax.axis_index("subcore") * nc + lax.axis_index("core")
        base = wid * WINDOW
        pltpu.sync_copy(i_hbm.at[pl.ds(base, WINDOW)], idx_v)
        pltpu.sync_copy(x_hbm.at[pl.ds(base, WINDOW)], rows_v)
        pltpu.async_copy(rows_v, o_hbm.at[idx_v], sem).wait()

    return k(x_flat, dest_1d)


def _combine(y_sorted, dest_1d):
    nc = _num_cores()

    @functools.partial(
        pl.kernel,
        out_type=jax.ShapeDtypeStruct((N, DIM), jnp.float32),
        mesh=_vmesh(),
        scratch_types=[pltpu.VMEM((WINDOW,), jnp.int32),
                       pltpu.VMEM((WINDOW, DIM), jnp.float32),
                       pltpu.SemaphoreType.DMA],
    )
    def k(y_hbm, i_hbm, o_hbm, idx_v, rows_v, sem):
        wid = lax.axis_index("subcore") * nc + lax.axis_index("core")
        base = wid * WINDOW
        pltpu.sync_copy(i_hbm.at[pl.ds(base, WINDOW)], idx_v)
        pltpu.async_copy(y_hbm.at[idx_v], rows_v, sem).wait()
        pltpu.sync_copy(rows_v, o_hbm.at[pl.ds(base, WINDOW)])

    return k(y_sorted, dest_1d)


def kernel(x, gate_w, w1, w2, w3):
    Bv, T, C = x.shape
    x_flat = x.reshape(T, C)
    gw_pad = jnp.zeros((LANES, C), gate_w.dtype).at[:E].set(gate_w)
    dest, start_t, tiles_t = _router(x_flat, gw_pad)
    dest_1d = dest.reshape(N)
    start_e = start_t.reshape(LANES)[:E]
    tiles_e = tiles_t.reshape(LANES)[:E]
    x_sorted = _dispatch(x_flat, dest_1d)
    y_sorted = _ffn(x_sorted, w1, w3, w2, start_e, tiles_e)
    out = _combine(y_sorted, dest_1d)
    return out.reshape(Bv, T, C)


# speedup vs baseline: 1.6162x; 1.6162x over previous
import functools

import jax
import jax.numpy as jnp
from jax import lax
from jax.experimental import pallas as pl
from jax.experimental.pallas import tpu as pltpu
from jax.experimental.pallas import tpu_sc as plsc

DIM = 768
HIDDEN = 2048
E = 8
N = 2048
TILE = 512
MAX_TILES = N // TILE + E - 1
PAD_N = MAX_TILES * TILE
LANES = 128
WINDOW = 64


def _shift_rows(c, k):
    return jnp.concatenate([jnp.zeros((k, c.shape[1]), c.dtype), c[:-k, :]], axis=0)


def _shift_lanes(c, k):
    return jnp.concatenate([jnp.zeros((c.shape[0], k), c.dtype), c[:, :-k]], axis=1)


def _router_body(x_ref, gw_ref, dest_ref, te_ref):
    x = x_ref[...]
    gw = gw_ref[...]
    logits = lax.dot_general(x, gw, (((1,), (1,)), ((), ())),
                             preferred_element_type=jnp.float32)
    col = lax.broadcasted_iota(jnp.int32, (N, LANES), 1)
    valid = col < E
    lm = jnp.where(valid, logits, -jnp.inf)
    m = jnp.max(lm, axis=1, keepdims=True)
    ex = jnp.exp(lm - m)
    p = ex / jnp.sum(ex, axis=1, keepdims=True)
    pmax = jnp.max(p, axis=1, keepdims=True)
    cand = jnp.where((p == pmax) & valid, col, LANES)
    eid = jnp.min(cand, axis=1, keepdims=True)
    onehot = (col == eid).astype(jnp.int32)

    c = onehot
    k = 1
    while k < N:
        c = c + _shift_rows(c, k)
        k *= 2
    counts = c[N - 1:N, :]
    rank = jnp.sum(c * onehot, axis=1, keepdims=True) - 1

    tiles = (counts + (TILE - 1)) // TILE
    cuminc = tiles
    k = 1
    while k < E:
        cuminc = cuminc + _shift_lanes(cuminc, k)
        k *= 2
    start = cuminc - tiles
    base = jnp.sum(onehot * (start * TILE), axis=1, keepdims=True)
    dest_ref[...] = base + rank

    rowi = lax.broadcasted_iota(jnp.int32, (LANES, LANES), 0)
    coli = lax.broadcasted_iota(jnp.int32, (LANES, LANES), 1)
    eqmat = (rowi == coli).astype(jnp.float32)
    counts_b = jnp.concatenate([counts.astype(jnp.float32)] * 8, axis=0)
    counts_col = lax.dot_general(eqmat, counts_b,
                                 (((1,), (1,)), ((), ())),
                                 preferred_element_type=jnp.float32)[:, 0:1]
    tiles_col = jnp.floor((counts_col + (TILE - 1)) / TILE)
    cum_col = tiles_col
    k = 1
    while k < E:
        cum_col = cum_col + jnp.concatenate(
            [jnp.zeros((k, 1), jnp.float32), cum_col[:-k, :]], axis=0)
        k *= 2
    start_col = cum_col - tiles_col
    colt = coli.astype(jnp.float32)
    rowe = rowi.astype(jnp.float32)
    mask = (colt >= start_col) & (colt < start_col + tiles_col)
    te = jnp.sum(jnp.where(mask, rowe, 0.0), axis=0, keepdims=True)
    te_ref[...] = te.astype(jnp.int32)


def _router(x_flat, gw_pad):
    return pl.pallas_call(
        _router_body,
        out_shape=(jax.ShapeDtypeStruct((N, 1), jnp.int32),
                   jax.ShapeDtypeStruct((1, LANES), jnp.int32)),
    )(x_flat, gw_pad)


def _ffn_body(te_ref, x_ref, w1_ref, w3_ref, w2_ref, y_ref):
    del te_ref
    x = x_ref[...].astype(jnp.bfloat16)
    a = lax.dot_general(x, w1_ref[0].astype(jnp.bfloat16), (((1,), (1,)), ((), ())),
                        preferred_element_type=jnp.float32)
    b = lax.dot_general(x, w3_ref[0].astype(jnp.bfloat16), (((1,), (1,)), ((), ())),
                        preferred_element_type=jnp.float32)
    h = (a * jax.nn.sigmoid(a) * b).astype(jnp.bfloat16)
    y_ref[...] = lax.dot_general(h, w2_ref[0].astype(jnp.bfloat16), (((1,), (1,)), ((), ())),
                                 preferred_element_type=jnp.float32)


def _ffn(x_sorted, w1, w3, w2, te):
    grid_spec = pltpu.PrefetchScalarGridSpec(
        num_scalar_prefetch=1,
        grid=(MAX_TILES,),
        in_specs=[
            pl.BlockSpec((TILE, DIM), lambda t, te: (t, 0)),
            pl.BlockSpec((1, HIDDEN, DIM), lambda t, te: (te[t], 0, 0)),
            pl.BlockSpec((1, HIDDEN, DIM), lambda t, te: (te[t], 0, 0)),
            pl.BlockSpec((1, DIM, HIDDEN), lambda t, te: (te[t], 0, 0)),
        ],
        out_specs=pl.BlockSpec((TILE, DIM), lambda t, te: (t, 0)),
    )
    return pl.pallas_call(
        _ffn_body,
        grid_spec=grid_spec,
        out_shape=jax.ShapeDtypeStruct((PAD_N, DIM), jnp.float32),
    )(te, x_sorted, w1, w3, w2)


@functools.cache
def _vmesh():
    return plsc.VectorSubcoreMesh(core_axis_name="core", subcore_axis_name="subcore")


@functools.cache
def _num_cores():
    return plsc.get_sparse_core_info().num_cores


def _dispatch(x_flat, dest_1d):
    nc = _num_cores()

    @functools.partial(
        pl.kernel,
        out_type=jax.ShapeDtypeStruct((PAD_N, DIM), jnp.float32),
        mesh=_vmesh(),
        scratch_types=[pltpu.VMEM((WINDOW,), jnp.int32),
                       pltpu.VMEM((WINDOW, DIM), jnp.float32),
                       pltpu.SemaphoreType.DMA],
    )
    def k(x_hbm, i_hbm, o_hbm, idx_v, rows_v, sem):
        wid = lax.axis_index("subcore") * nc + lax.axis_index("core")
        base = wid * WINDOW
        pltpu.sync_copy(i_hbm.at[pl.ds(base, WINDOW)], idx_v)
        pltpu.sync_copy(x_hbm.at[pl.ds(base, WINDOW)], rows_v)
        pltpu.async_copy(rows_v, o_hbm.at[idx_v], sem).wait()

    return k(x_flat, dest_1d)


def _combine(y_sorted, dest_1d):
    nc = _num_cores()

    @functools.partial(
        pl.kernel,
        out_type=jax.ShapeDtypeStruct((N, DIM), jnp.float32),
        mesh=_vmesh(),
        scratch_types=[pltpu.VMEM((WINDOW,), jnp.int32),
                       pltpu.VMEM((WINDOW, DIM), jnp.float32),
                       pltpu.SemaphoreType.DMA],
    )
    def k(y_hbm, i_hbm, o_hbm, idx_v, rows_v, sem):
        wid = lax.axis_index("subcore") * nc + lax.axis_index("core")
        base = wid * WINDOW
        pltpu.sync_copy(i_hbm.at[pl.ds(base, WINDOW)], idx_v)
        pltpu.async_copy(y_hbm.at[idx_v], rows_v, sem).wait()
        pltpu.sync_copy(rows_v, o_hbm.at[pl.ds(base, WINDOW)])

    return k(y_sorted, dest_1d)


def kernel(x, gate_w, w1, w2, w3):
    Bv, T, C = x.shape
    x_flat = x.reshape(T, C)
    gw_pad = jnp.zeros((LANES, C), gate_w.dtype).at[:E].set(gate_w)
    dest, te = _router(x_flat, gw_pad)
    dest_1d = dest.reshape(N)
    te_flat = te.reshape(LANES)[:MAX_TILES]
    x_sorted = _dispatch(x_flat, dest_1d)
    y_sorted = _ffn(x_sorted, w1, w3, w2, te_flat)
    out = _combine(y_sorted, dest_1d)
    return out.reshape(Bv, T, C)


# speedup vs baseline: 1.6375x; 1.0132x over previous
import functools

import jax
import jax.numpy as jnp
from jax import lax
from jax.experimental import pallas as pl
from jax.experimental.pallas import tpu as pltpu
from jax.experimental.pallas import tpu_sc as plsc

DIM = 768
HIDDEN = 2048
E = 8
N = 2048
TILE = 512
MAX_TILES = N // TILE + E - 1
PAD_N = MAX_TILES * TILE
LANES = 128
WINDOW = 64


def _shift_rows(c, k):
    return jnp.concatenate([jnp.zeros((k, c.shape[1]), c.dtype), c[:-k, :]], axis=0)


def _shift_lanes(c, k):
    return jnp.concatenate([jnp.zeros((c.shape[0], k), c.dtype), c[:, :-k]], axis=1)


def _router_body(x_ref, gw_ref, dest_ref, te_ref):
    x = x_ref[...]
    gw = gw_ref[...]
    logits = lax.dot_general(x, gw, (((1,), (1,)), ((), ())),
                             preferred_element_type=jnp.float32)
    col = lax.broadcasted_iota(jnp.int32, (N, LANES), 1)
    valid = col < E
    lm = jnp.where(valid, logits, -jnp.inf)
    m = jnp.max(lm, axis=1, keepdims=True)
    ex = jnp.exp(lm - m)
    p = ex / jnp.sum(ex, axis=1, keepdims=True)
    pmax = jnp.max(p, axis=1, keepdims=True)
    cand = jnp.where((p == pmax) & valid, col, LANES)
    eid = jnp.min(cand, axis=1, keepdims=True)
    onehot = (col == eid).astype(jnp.int32)

    c = onehot
    k = 1
    while k < N:
        c = c + _shift_rows(c, k)
        k *= 2
    counts = c[N - 1:N, :]
    rank = jnp.sum(c * onehot, axis=1, keepdims=True) - 1

    tiles = (counts + (TILE - 1)) // TILE
    cuminc = tiles
    k = 1
    while k < E:
        cuminc = cuminc + _shift_lanes(cuminc, k)
        k *= 2
    start = cuminc - tiles
    base = jnp.sum(onehot * (start * TILE), axis=1, keepdims=True)
    dest_ref[...] = base + rank

    rowi = lax.broadcasted_iota(jnp.int32, (LANES, LANES), 0)
    coli = lax.broadcasted_iota(jnp.int32, (LANES, LANES), 1)
    eqmat = (rowi == coli).astype(jnp.float32)
    counts_b = jnp.concatenate([counts.astype(jnp.float32)] * 8, axis=0)
    counts_col = lax.dot_general(eqmat, counts_b,
                                 (((1,), (1,)), ((), ())),
                                 preferred_element_type=jnp.float32)[:, 0:1]
    tiles_col = jnp.floor((counts_col + (TILE - 1)) / TILE)
    cum_col = tiles_col
    k = 1
    while k < E:
        cum_col = cum_col + jnp.concatenate(
            [jnp.zeros((k, 1), jnp.float32), cum_col[:-k, :]], axis=0)
        k *= 2
    start_col = cum_col - tiles_col
    colt = coli.astype(jnp.float32)
    rowe = rowi.astype(jnp.float32)
    mask = (colt >= start_col) & (colt < start_col + tiles_col)
    te = jnp.sum(jnp.where(mask, rowe, 0.0), axis=0, keepdims=True)
    covered = jnp.sum(jnp.where(mask, 1.0, 0.0), axis=0, keepdims=True)
    te = te + (1.0 - covered) * (E - 1)
    te_ref[...] = te.astype(jnp.int32)


def _router(x_flat, gw_pad):
    return pl.pallas_call(
        _router_body,
        out_shape=(jax.ShapeDtypeStruct((N, 1), jnp.int32),
                   jax.ShapeDtypeStruct((1, LANES), jnp.int32)),
    )(x_flat, gw_pad)


def _ffn_body(te_ref, x_ref, w1_ref, w3_ref, w2_ref, y_ref):
    del te_ref
    x = x_ref[...].astype(jnp.bfloat16)
    a = lax.dot_general(x, w1_ref[0].astype(jnp.bfloat16), (((1,), (1,)), ((), ())),
                        preferred_element_type=jnp.float32)
    b = lax.dot_general(x, w3_ref[0].astype(jnp.bfloat16), (((1,), (1,)), ((), ())),
                        preferred_element_type=jnp.float32)
    h = (a * jax.nn.sigmoid(a) * b).astype(jnp.bfloat16)
    y_ref[...] = lax.dot_general(h, w2_ref[0].astype(jnp.bfloat16), (((1,), (1,)), ((), ())),
                                 preferred_element_type=jnp.float32)


def _ffn(x_sorted, w1, w3, w2, te):
    grid_spec = pltpu.PrefetchScalarGridSpec(
        num_scalar_prefetch=1,
        grid=(MAX_TILES,),
        in_specs=[
            pl.BlockSpec((TILE, DIM), lambda t, te: (t, 0)),
            pl.BlockSpec((1, HIDDEN, DIM), lambda t, te: (te[t], 0, 0)),
            pl.BlockSpec((1, HIDDEN, DIM), lambda t, te: (te[t], 0, 0)),
            pl.BlockSpec((1, DIM, HIDDEN), lambda t, te: (te[t], 0, 0)),
        ],
        out_specs=pl.BlockSpec((TILE, DIM), lambda t, te: (t, 0)),
    )
    return pl.pallas_call(
        _ffn_body,
        grid_spec=grid_spec,
        out_shape=jax.ShapeDtypeStruct((PAD_N, DIM), jnp.float32),
    )(te, x_sorted, w1, w3, w2)


@functools.cache
def _vmesh():
    return plsc.VectorSubcoreMesh(core_axis_name="core", subcore_axis_name="subcore")


@functools.cache
def _num_cores():
    return plsc.get_sparse_core_info().num_cores


def _dispatch(x_flat, dest_1d):
    nc = _num_cores()

    @functools.partial(
        pl.kernel,
        out_type=jax.ShapeDtypeStruct((PAD_N, DIM), jnp.float32),
        mesh=_vmesh(),
        scratch_types=[pltpu.VMEM((WINDOW,), jnp.int32),
                       pltpu.VMEM((WINDOW, DIM), jnp.float32),
                       pltpu.SemaphoreType.DMA],
    )
    def k(x_hbm, i_hbm, o_hbm, idx_v, rows_v, sem):
        wid = lax.axis_index("subcore") * nc + lax.axis_index("core")
        base = wid * WINDOW
        pltpu.sync_copy(i_hbm.at[pl.ds(base, WINDOW)], idx_v)
        pltpu.sync_copy(x_hbm.at[pl.ds(base, WINDOW)], rows_v)
        pltpu.async_copy(rows_v, o_hbm.at[idx_v], sem).wait()

    return k(x_flat, dest_1d)


def _combine(y_sorted, dest_1d):
    nc = _num_cores()

    @functools.partial(
        pl.kernel,
        out_type=jax.ShapeDtypeStruct((N, DIM), jnp.float32),
        mesh=_vmesh(),
        scratch_types=[pltpu.VMEM((WINDOW,), jnp.int32),
                       pltpu.VMEM((WINDOW, DIM), jnp.float32),
                       pltpu.SemaphoreType.DMA],
    )
    def k(y_hbm, i_hbm, o_hbm, idx_v, rows_v, sem):
        wid = lax.axis_index("subcore") * nc + lax.axis_index("core")
        base = wid * WINDOW
        pltpu.sync_copy(i_hbm.at[pl.ds(base, WINDOW)], idx_v)
        pltpu.async_copy(y_hbm.at[idx_v], rows_v, sem).wait()
        pltpu.sync_copy(rows_v, o_hbm.at[pl.ds(base, WINDOW)])

    return k(y_sorted, dest_1d)


def kernel(x, gate_w, w1, w2, w3):
    Bv, T, C = x.shape
    x_flat = x.reshape(T, C)
    gw_pad = jnp.zeros((LANES, C), gate_w.dtype).at[:E].set(gate_w)
    dest, te = _router(x_flat, gw_pad)
    dest_1d = dest.reshape(N)
    te_flat = te.reshape(LANES)[:MAX_TILES]
    x_sorted = _dispatch(x_flat, dest_1d)
    y_sorted = _ffn(x_sorted, w1, w3, w2, te_flat)
    out = _combine(y_sorted, dest_1d)
    return out.reshape(Bv, T, C)


# speedup vs baseline: 1.6413x; 1.0023x over previous
import functools

import jax
import jax.numpy as jnp
from jax import lax
from jax.experimental import pallas as pl
from jax.experimental.pallas import tpu as pltpu
from jax.experimental.pallas import tpu_sc as plsc

DIM = 768
HIDDEN = 2048
E = 8
N = 2048
TILE = 512
MAX_TILES = N // TILE + E - 1
PAD_N = MAX_TILES * TILE
LANES = 128
WINDOW = 64


def _shift_rows(c, k):
    return jnp.concatenate([jnp.zeros((k, c.shape[1]), c.dtype), c[:-k, :]], axis=0)


def _shift_lanes(c, k):
    return jnp.concatenate([jnp.zeros((c.shape[0], k), c.dtype), c[:, :-k]], axis=1)


def _router_body(x_ref, gw_ref, dest_ref, te_ref):
    x = x_ref[...]
    gw = gw_ref[...]
    logits = lax.dot_general(x, gw, (((1,), (1,)), ((), ())),
                             preferred_element_type=jnp.float32)
    col = lax.broadcasted_iota(jnp.int32, (N, LANES), 1)
    valid = col < E
    lm = jnp.where(valid, logits, -jnp.inf)
    m = jnp.max(lm, axis=1, keepdims=True)
    ex = jnp.exp(lm - m)
    p = ex / jnp.sum(ex, axis=1, keepdims=True)
    pmax = jnp.max(p, axis=1, keepdims=True)
    cand = jnp.where((p == pmax) & valid, col, LANES)
    eid = jnp.min(cand, axis=1, keepdims=True)
    onehot = (col == eid).astype(jnp.int32)

    c = onehot
    k = 1
    while k < N:
        c = c + _shift_rows(c, k)
        k *= 2
    counts = c[N - 1:N, :]
    rank = jnp.sum(c * onehot, axis=1, keepdims=True) - 1

    tiles = (counts + (TILE - 1)) // TILE
    cuminc = tiles
    k = 1
    while k < E:
        cuminc = cuminc + _shift_lanes(cuminc, k)
        k *= 2
    start = cuminc - tiles
    base = jnp.sum(onehot * (start * TILE), axis=1, keepdims=True)
    dest_ref[...] = base + rank

    rowi = lax.broadcasted_iota(jnp.int32, (LANES, LANES), 0)
    coli = lax.broadcasted_iota(jnp.int32, (LANES, LANES), 1)
    eqmat = (rowi == coli).astype(jnp.float32)
    counts_b = jnp.concatenate([counts.astype(jnp.float32)] * 8, axis=0)
    counts_col = lax.dot_general(eqmat, counts_b,
                                 (((1,), (1,)), ((), ())),
                                 preferred_element_type=jnp.float32)[:, 0:1]
    tiles_col = jnp.floor((counts_col + (TILE - 1)) / TILE)
    cum_col = tiles_col
    k = 1
    while k < E:
        cum_col = cum_col + jnp.concatenate(
            [jnp.zeros((k, 1), jnp.float32), cum_col[:-k, :]], axis=0)
        k *= 2
    start_col = cum_col - tiles_col
    colt = coli.astype(jnp.float32)
    rowe = rowi.astype(jnp.float32)
    mask = (colt >= start_col) & (colt < start_col + tiles_col)
    te = jnp.sum(jnp.where(mask, rowe, 0.0), axis=0, keepdims=True)
    covered = jnp.sum(jnp.where(mask, 1.0, 0.0), axis=0, keepdims=True)
    te = te + (1.0 - covered) * (E - 1)
    te_ref[...] = te.astype(jnp.int32)


def _router(x_flat, gw_pad):
    return pl.pallas_call(
        _router_body,
        out_shape=(jax.ShapeDtypeStruct((N, 1), jnp.int32),
                   jax.ShapeDtypeStruct((1, LANES), jnp.int32)),
    )(x_flat, gw_pad)


def _ffn_body(te_ref, x_ref, w1_ref, w3_ref, w2_ref, y_ref):
    del te_ref
    x = x_ref[...]
    a = lax.dot_general(x, w1_ref[0], (((1,), (1,)), ((), ())),
                        preferred_element_type=jnp.float32)
    b = lax.dot_general(x, w3_ref[0], (((1,), (1,)), ((), ())),
                        preferred_element_type=jnp.float32)
    h = a * jax.nn.sigmoid(a) * b
    y_ref[...] = lax.dot_general(h, w2_ref[0], (((1,), (1,)), ((), ())),
                                 preferred_element_type=jnp.float32)


def _ffn(x_sorted, w1, w3, w2, te):
    grid_spec = pltpu.PrefetchScalarGridSpec(
        num_scalar_prefetch=1,
        grid=(MAX_TILES,),
        in_specs=[
            pl.BlockSpec((TILE, DIM), lambda t, te: (t, 0)),
            pl.BlockSpec((1, HIDDEN, DIM), lambda t, te: (te[t], 0, 0)),
            pl.BlockSpec((1, HIDDEN, DIM), lambda t, te: (te[t], 0, 0)),
            pl.BlockSpec((1, DIM, HIDDEN), lambda t, te: (te[t], 0, 0)),
        ],
        out_specs=pl.BlockSpec((TILE, DIM), lambda t, te: (t, 0)),
    )
    return pl.pallas_call(
        _ffn_body,
        grid_spec=grid_spec,
        out_shape=jax.ShapeDtypeStruct((PAD_N, DIM), jnp.float32),
    )(te, x_sorted, w1, w3, w2)


@functools.cache
def _vmesh():
    return plsc.VectorSubcoreMesh(core_axis_name="core", subcore_axis_name="subcore")


@functools.cache
def _num_cores():
    return plsc.get_sparse_core_info().num_cores


def _dispatch(x_flat, dest_1d):
    nc = _num_cores()

    @functools.partial(
        pl.kernel,
        out_type=jax.ShapeDtypeStruct((PAD_N, DIM), jnp.float32),
        mesh=_vmesh(),
        scratch_types=[pltpu.VMEM((WINDOW,), jnp.int32),
                       pltpu.VMEM((WINDOW, DIM), jnp.float32),
                       pltpu.SemaphoreType.DMA],
    )
    def k(x_hbm, i_hbm, o_hbm, idx_v, rows_v, sem):
        wid = lax.axis_index("subcore") * nc + lax.axis_index("core")
        base = wid * WINDOW
        pltpu.sync_copy(i_hbm.at[pl.ds(base, WINDOW)], idx_v)
        pltpu.sync_copy(x_hbm.at[pl.ds(base, WINDOW)], rows_v)
        pltpu.async_copy(rows_v, o_hbm.at[idx_v], sem).wait()

    return k(x_flat, dest_1d)


def _combine(y_sorted, dest_1d):
    nc = _num_cores()

    @functools.partial(
        pl.kernel,
        out_type=jax.ShapeDtypeStruct((N, DIM), jnp.float32),
        mesh=_vmesh(),
        scratch_types=[pltpu.VMEM((WINDOW,), jnp.int32),
                       pltpu.VMEM((WINDOW, DIM), jnp.float32),
                       pltpu.SemaphoreType.DMA],
    )
    def k(y_hbm, i_hbm, o_hbm, idx_v, rows_v, sem):
        wid = lax.axis_index("subcore") * nc + lax.axis_index("core")
        base = wid * WINDOW
        pltpu.sync_copy(i_hbm.at[pl.ds(base, WINDOW)], idx_v)
        pltpu.async_copy(y_hbm.at[idx_v], rows_v, sem).wait()
        pltpu.sync_copy(rows_v, o_hbm.at[pl.ds(base, WINDOW)])

    return k(y_sorted, dest_1d)


def kernel(x, gate_w, w1, w2, w3):
    Bv, T, C = x.shape
    x_flat = x.reshape(T, C)
    gw_pad = jnp.zeros((LANES, C), gate_w.dtype).at[:E].set(gate_w)
    dest, te = _router(x_flat, gw_pad)
    dest_1d = dest.reshape(N)
    te_flat = te.reshape(LANES)[:MAX_TILES]
    x_sorted = _dispatch(x_flat, dest_1d)
    y_sorted = _ffn(x_sorted, w1, w3, w2, te_flat)
    out = _combine(y_sorted, dest_1d)
    return out.reshape(Bv, T, C)


# speedup vs baseline: 1.8422x; 1.1224x over previous
import functools

import jax
import jax.numpy as jnp
from jax import lax
from jax.experimental import pallas as pl
from jax.experimental.pallas import tpu as pltpu
from jax.experimental.pallas import tpu_sc as plsc

DIM = 768
HIDDEN = 2048
E = 8
N = 2048
TILE = 512
MAX_TILES = N // TILE + E - 1
PAD_N = MAX_TILES * TILE
LANES = 128
WINDOW = 64


def _shift_rows(c, k):
    return jnp.concatenate([jnp.zeros((k, c.shape[1]), c.dtype), c[:-k, :]], axis=0)


def _shift_lanes(c, k):
    return jnp.concatenate([jnp.zeros((c.shape[0], k), c.dtype), c[:, :-k]], axis=1)


def _router_body(x_ref, gw_ref, dest_ref, te_ref):
    x = x_ref[...]
    gw = gw_ref[...]
    logits = lax.dot_general(x, gw, (((1,), (1,)), ((), ())),
                             preferred_element_type=jnp.float32)
    col = lax.broadcasted_iota(jnp.int32, (N, LANES), 1)
    valid = col < E
    lm = jnp.where(valid, logits, -jnp.inf)
    m = jnp.max(lm, axis=1, keepdims=True)
    ex = jnp.exp(lm - m)
    p = ex / jnp.sum(ex, axis=1, keepdims=True)
    pmax = jnp.max(p, axis=1, keepdims=True)
    cand = jnp.where((p == pmax) & valid, col, LANES)
    eid = jnp.min(cand, axis=1, keepdims=True)
    onehot = (col == eid).astype(jnp.int32)

    c = onehot
    k = 1
    while k < N:
        c = c + _shift_rows(c, k)
        k *= 2
    counts = c[N - 1:N, :]
    rank = jnp.sum(c * onehot, axis=1, keepdims=True) - 1

    tiles = (counts + (TILE - 1)) // TILE
    cuminc = tiles
    k = 1
    while k < E:
        cuminc = cuminc + _shift_lanes(cuminc, k)
        k *= 2
    start = cuminc - tiles
    base = jnp.sum(onehot * (start * TILE), axis=1, keepdims=True)
    dest_ref[...] = base + rank

    rowi = lax.broadcasted_iota(jnp.int32, (LANES, LANES), 0)
    coli = lax.broadcasted_iota(jnp.int32, (LANES, LANES), 1)
    eqmat = (rowi == coli).astype(jnp.float32)
    counts_b = jnp.concatenate([counts.astype(jnp.float32)] * 8, axis=0)
    counts_col = lax.dot_general(eqmat, counts_b,
                                 (((1,), (1,)), ((), ())),
                                 preferred_element_type=jnp.float32)[:, 0:1]
    tiles_col = jnp.floor((counts_col + (TILE - 1)) / TILE)
    cum_col = tiles_col
    k = 1
    while k < E:
        cum_col = cum_col + jnp.concatenate(
            [jnp.zeros((k, 1), jnp.float32), cum_col[:-k, :]], axis=0)
        k *= 2
    start_col = cum_col - tiles_col
    colt = coli.astype(jnp.float32)
    rowe = rowi.astype(jnp.float32)
    mask = (colt >= start_col) & (colt < start_col + tiles_col)
    te = jnp.sum(jnp.where(mask, rowe, 0.0), axis=0, keepdims=True)
    covered = jnp.sum(jnp.where(mask, 1.0, 0.0), axis=0, keepdims=True)
    te = te + (1.0 - covered) * (E - 1)
    lane = coli[0:1, :]
    total_b = jnp.sum(jnp.where(lane == E - 1, cuminc.astype(jnp.float32), 0.0),
                      axis=1, keepdims=True)
    te = jnp.where(lane == MAX_TILES, total_b, te)
    te_ref[...] = te.astype(jnp.int32)


def _router(x_flat, gw_pad):
    return pl.pallas_call(
        _router_body,
        out_shape=(jax.ShapeDtypeStruct((N, 1), jnp.int32),
                   jax.ShapeDtypeStruct((1, LANES), jnp.int32)),
    )(x_flat, gw_pad)


def _ffn_body(te_ref, x_ref, w1_ref, w3_ref, w2_ref, y_ref):
    @pl.when(pl.program_id(0) < te_ref[MAX_TILES])
    def _():
        x = x_ref[...].astype(jnp.bfloat16)
        a = lax.dot_general(x, w1_ref[0].astype(jnp.bfloat16), (((1,), (1,)), ((), ())),
                            preferred_element_type=jnp.float32)
        b = lax.dot_general(x, w3_ref[0].astype(jnp.bfloat16), (((1,), (1,)), ((), ())),
                            preferred_element_type=jnp.float32)
        h = (a * jax.nn.sigmoid(a) * b).astype(jnp.bfloat16)
        y_ref[...] = lax.dot_general(h, w2_ref[0].astype(jnp.bfloat16), (((1,), (1,)), ((), ())),
                                     preferred_element_type=jnp.float32)


def _ffn(x_sorted, w1, w3, w2, te):
    grid_spec = pltpu.PrefetchScalarGridSpec(
        num_scalar_prefetch=1,
        grid=(MAX_TILES,),
        in_specs=[
            pl.BlockSpec((TILE, DIM), lambda t, te: (t, 0)),
            pl.BlockSpec((1, HIDDEN, DIM), lambda t, te: (te[t], 0, 0)),
            pl.BlockSpec((1, HIDDEN, DIM), lambda t, te: (te[t], 0, 0)),
            pl.BlockSpec((1, DIM, HIDDEN), lambda t, te: (te[t], 0, 0)),
        ],
        out_specs=pl.BlockSpec((TILE, DIM), lambda t, te: (t, 0)),
    )
    return pl.pallas_call(
        _ffn_body,
        grid_spec=grid_spec,
        out_shape=jax.ShapeDtypeStruct((PAD_N, DIM), jnp.float32),
    )(te, x_sorted, w1, w3, w2)


@functools.cache
def _vmesh():
    return plsc.VectorSubcoreMesh(core_axis_name="core", subcore_axis_name="subcore")


@functools.cache
def _num_cores():
    return plsc.get_sparse_core_info().num_cores


def _dispatch(x_flat, dest_1d):
    nc = _num_cores()

    @functools.partial(
        pl.kernel,
        out_type=jax.ShapeDtypeStruct((PAD_N, DIM), jnp.float32),
        mesh=_vmesh(),
        scratch_types=[pltpu.VMEM((WINDOW,), jnp.int32),
                       pltpu.VMEM((WINDOW, DIM), jnp.float32),
                       pltpu.SemaphoreType.DMA],
    )
    def k(x_hbm, i_hbm, o_hbm, idx_v, rows_v, sem):
        wid = lax.axis_index("subcore") * nc + lax.axis_index("core")
        base = wid * WINDOW
        pltpu.sync_copy(i_hbm.at[pl.ds(base, WINDOW)], idx_v)
        pltpu.sync_copy(x_hbm.at[pl.ds(base, WINDOW)], rows_v)
        pltpu.async_copy(rows_v, o_hbm.at[idx_v], sem).wait()

    return k(x_flat, dest_1d)


def _combine(y_sorted, dest_1d):
    nc = _num_cores()

    @functools.partial(
        pl.kernel,
        out_type=jax.ShapeDtypeStruct((N, DIM), jnp.float32),
        mesh=_vmesh(),
        scratch_types=[pltpu.VMEM((WINDOW,), jnp.int32),
                       pltpu.VMEM((WINDOW, DIM), jnp.float32),
                       pltpu.SemaphoreType.DMA],
    )
    def k(y_hbm, i_hbm, o_hbm, idx_v, rows_v, sem):
        wid = lax.axis_index("subcore") * nc + lax.axis_index("core")
        base = wid * WINDOW
        pltpu.sync_copy(i_hbm.at[pl.ds(base, WINDOW)], idx_v)
        pltpu.async_copy(y_hbm.at[idx_v], rows_v, sem).wait()
        pltpu.sync_copy(rows_v, o_hbm.at[pl.ds(base, WINDOW)])

    return k(y_sorted, dest_1d)


def kernel(x, gate_w, w1, w2, w3):
    Bv, T, C = x.shape
    x_flat = x.reshape(T, C)
    gw_pad = jnp.zeros((LANES, C), gate_w.dtype).at[:E].set(gate_w)
    dest, te = _router(x_flat, gw_pad)
    dest_1d = dest.reshape(N)
    te_flat = te.reshape(LANES)[:MAX_TILES + 1]
    x_sorted = _dispatch(x_flat, dest_1d)
    y_sorted = _ffn(x_sorted, w1, w3, w2, te_flat)
    out = _combine(y_sorted, dest_1d)
    return out.reshape(Bv, T, C)


# speedup vs baseline: 1.8505x; 1.0045x over previous
import functools

import jax
import jax.numpy as jnp
from jax import lax
from jax.experimental import pallas as pl
from jax.experimental.pallas import tpu as pltpu
from jax.experimental.pallas import tpu_sc as plsc

DIM = 768
HIDDEN = 2048
E = 8
N = 2048
TILE = 512
MAX_TILES = N // TILE + E - 1
PAD_N = MAX_TILES * TILE
LANES = 128
WINDOW = 64


def _shift_rows(c, k):
    return jnp.concatenate([jnp.zeros((k, c.shape[1]), c.dtype), c[:-k, :]], axis=0)


def _shift_lanes(c, k):
    return jnp.concatenate([jnp.zeros((c.shape[0], k), c.dtype), c[:, :-k]], axis=1)


def _router_body(x_ref, gw_ref, dest_ref, te_ref):
    x = x_ref[...]
    gw = gw_ref[...]
    logits = lax.dot_general(x, gw, (((1,), (1,)), ((), ())),
                             preferred_element_type=jnp.float32)
    col = lax.broadcasted_iota(jnp.int32, (N, LANES), 1)
    valid = col < E
    lm = jnp.where(valid, logits, -jnp.inf)
    m = jnp.max(lm, axis=1, keepdims=True)
    ex = jnp.exp(lm - m)
    p = ex / jnp.sum(ex, axis=1, keepdims=True)
    pmax = jnp.max(p, axis=1, keepdims=True)
    cand = jnp.where((p == pmax) & valid, col, LANES)
    eid = jnp.min(cand, axis=1, keepdims=True)
    onehot = (col == eid).astype(jnp.int32)

    c = onehot
    k = 1
    while k < N:
        c = c + _shift_rows(c, k)
        k *= 2
    counts = c[N - 1:N, :]
    rank = jnp.sum(c * onehot, axis=1, keepdims=True) - 1

    tiles = (counts + (TILE - 1)) // TILE
    cuminc = tiles
    k = 1
    while k < E:
        cuminc = cuminc + _shift_lanes(cuminc, k)
        k *= 2
    start = cuminc - tiles
    base = jnp.sum(onehot * (start * TILE), axis=1, keepdims=True)
    dest_ref[...] = base + rank

    rowi = lax.broadcasted_iota(jnp.int32, (LANES, LANES), 0)
    coli = lax.broadcasted_iota(jnp.int32, (LANES, LANES), 1)
    eqmat = (rowi == coli).astype(jnp.float32)
    counts_b = jnp.concatenate([counts.astype(jnp.float32)] * 8, axis=0)
    counts_col = lax.dot_general(eqmat, counts_b,
                                 (((1,), (1,)), ((), ())),
                                 preferred_element_type=jnp.float32)[:, 0:1]
    tiles_col = jnp.floor((counts_col + (TILE - 1)) / TILE)
    cum_col = tiles_col
    k = 1
    while k < E:
        cum_col = cum_col + jnp.concatenate(
            [jnp.zeros((k, 1), jnp.float32), cum_col[:-k, :]], axis=0)
        k *= 2
    start_col = cum_col - tiles_col
    colt = coli.astype(jnp.float32)
    rowe = rowi.astype(jnp.float32)
    mask = (colt >= start_col) & (colt < start_col + tiles_col)
    te = jnp.sum(jnp.where(mask, rowe, 0.0), axis=0, keepdims=True)
    covered = jnp.sum(jnp.where(mask, 1.0, 0.0), axis=0, keepdims=True)
    te = te + (1.0 - covered) * (E - 1)
    lane = coli[0:1, :]
    total_b = jnp.sum(jnp.where(lane == E - 1, cuminc.astype(jnp.float32), 0.0),
                      axis=1, keepdims=True)
    te = jnp.where(lane == MAX_TILES, total_b, te)
    te_ref[...] = te.astype(jnp.int32)


def _router(x_flat, gw_pad):
    return pl.pallas_call(
        _router_body,
        out_shape=(jax.ShapeDtypeStruct((N, 1), jnp.int32),
                   jax.ShapeDtypeStruct((1, LANES), jnp.int32)),
    )(x_flat, gw_pad)


def _ffn_body(te_ref, x_ref, w1_ref, w3_ref, w2_ref, y_ref):
    @pl.when(pl.program_id(0) < te_ref[MAX_TILES])
    def _():
        x = x_ref[...].astype(jnp.bfloat16)
        a = lax.dot_general(x, w1_ref[0].astype(jnp.bfloat16), (((1,), (1,)), ((), ())),
                            preferred_element_type=jnp.float32)
        b = lax.dot_general(x, w3_ref[0].astype(jnp.bfloat16), (((1,), (1,)), ((), ())),
                            preferred_element_type=jnp.float32)
        h = (a * jax.nn.sigmoid(a) * b).astype(jnp.bfloat16)
        y_ref[...] = lax.dot_general(h, w2_ref[0].astype(jnp.bfloat16), (((1,), (1,)), ((), ())),
                                     preferred_element_type=jnp.float32)


def _ffn(x_sorted, w1, w3, w2, te):
    grid_spec = pltpu.PrefetchScalarGridSpec(
        num_scalar_prefetch=1,
        grid=(MAX_TILES,),
        in_specs=[
            pl.BlockSpec((TILE, DIM), lambda t, te: (t, 0)),
            pl.BlockSpec((1, HIDDEN, DIM), lambda t, te: (te[t], 0, 0)),
            pl.BlockSpec((1, HIDDEN, DIM), lambda t, te: (te[t], 0, 0)),
            pl.BlockSpec((1, DIM, HIDDEN), lambda t, te: (te[t], 0, 0)),
        ],
        out_specs=pl.BlockSpec((TILE, DIM), lambda t, te: (t, 0)),
    )
    return pl.pallas_call(
        _ffn_body,
        grid_spec=grid_spec,
        out_shape=jax.ShapeDtypeStruct((PAD_N, DIM), jnp.float32),
    )(te, x_sorted, w1, w3, w2)


@functools.cache
def _vmesh():
    return plsc.VectorSubcoreMesh(core_axis_name="core", subcore_axis_name="subcore")


@functools.cache
def _num_cores():
    return plsc.get_sparse_core_info().num_cores


def _dispatch(x_flat, dest_1d):
    nc = _num_cores()

    @functools.partial(
        pl.kernel,
        out_type=jax.ShapeDtypeStruct((PAD_N, DIM), jnp.float32),
        mesh=_vmesh(),
        scratch_types=[pltpu.VMEM((WINDOW,), jnp.int32),
                       pltpu.VMEM((WINDOW, DIM), jnp.float32),
                       pltpu.SemaphoreType.DMA,
                       pltpu.SemaphoreType.DMA],
    )
    def k(x_hbm, i_hbm, o_hbm, idx_v, rows_v, sem, sem2):
        wid = lax.axis_index("subcore") * nc + lax.axis_index("core")
        base = wid * WINDOW
        cp_i = pltpu.make_async_copy(i_hbm.at[pl.ds(base, WINDOW)], idx_v, sem2)
        cp_x = pltpu.make_async_copy(x_hbm.at[pl.ds(base, WINDOW)], rows_v, sem)
        cp_i.start()
        cp_x.start()
        cp_i.wait()
        cp_x.wait()
        pltpu.async_copy(rows_v, o_hbm.at[idx_v], sem).wait()

    return k(x_flat, dest_1d)


def _combine(y_sorted, dest_1d):
    nc = _num_cores()

    @functools.partial(
        pl.kernel,
        out_type=jax.ShapeDtypeStruct((N, DIM), jnp.float32),
        mesh=_vmesh(),
        scratch_types=[pltpu.VMEM((WINDOW,), jnp.int32),
                       pltpu.VMEM((WINDOW, DIM), jnp.float32),
                       pltpu.SemaphoreType.DMA],
    )
    def k(y_hbm, i_hbm, o_hbm, idx_v, rows_v, sem):
        wid = lax.axis_index("subcore") * nc + lax.axis_index("core")
        base = wid * WINDOW
        pltpu.sync_copy(i_hbm.at[pl.ds(base, WINDOW)], idx_v)
        pltpu.async_copy(y_hbm.at[idx_v], rows_v, sem).wait()
        pltpu.sync_copy(rows_v, o_hbm.at[pl.ds(base, WINDOW)])

    return k(y_sorted, dest_1d)


def kernel(x, gate_w, w1, w2, w3):
    Bv, T, C = x.shape
    x_flat = x.reshape(T, C)
    gw_pad = jnp.zeros((LANES, C), gate_w.dtype).at[:E].set(gate_w)
    dest, te = _router(x_flat, gw_pad)
    dest_1d = dest.reshape(N)
    te_flat = te.reshape(LANES)[:MAX_TILES + 1]
    x_sorted = _dispatch(x_flat, dest_1d)
    y_sorted = _ffn(x_sorted, w1, w3, w2, te_flat)
    out = _combine(y_sorted, dest_1d)
    return out.reshape(Bv, T, C)


# speedup vs baseline: 1.8714x; 1.0113x over previous
import functools

import jax
import jax.numpy as jnp
from jax import lax
from jax.experimental import pallas as pl
from jax.experimental.pallas import tpu as pltpu
from jax.experimental.pallas import tpu_sc as plsc

DIM = 768
HIDDEN = 2048
E = 8
N = 2048
TILE = 512
MAX_TILES = N // TILE + E - 1
PAD_N = MAX_TILES * TILE
LANES = 128
WINDOW = 64


def _shift_rows(c, k):
    return jnp.concatenate([jnp.zeros((k, c.shape[1]), c.dtype), c[:-k, :]], axis=0)


def _shift_lanes(c, k):
    return jnp.concatenate([jnp.zeros((c.shape[0], k), c.dtype), c[:, :-k]], axis=1)


def _router_body(x_ref, gw_ref, dest_ref, te_ref):
    x = x_ref[...]
    gw = jnp.concatenate(
        [gw_ref[...], jnp.zeros((LANES - E, DIM), jnp.float32)], axis=0)
    logits = lax.dot_general(x, gw, (((1,), (1,)), ((), ())),
                             preferred_element_type=jnp.float32)
    col = lax.broadcasted_iota(jnp.int32, (N, LANES), 1)
    valid = col < E
    lm = jnp.where(valid, logits, -jnp.inf)
    m = jnp.max(lm, axis=1, keepdims=True)
    ex = jnp.exp(lm - m)
    p = ex / jnp.sum(ex, axis=1, keepdims=True)
    pmax = jnp.max(p, axis=1, keepdims=True)
    cand = jnp.where((p == pmax) & valid, col, LANES)
    eid = jnp.min(cand, axis=1, keepdims=True)
    onehot = (col == eid).astype(jnp.int32)

    c = onehot
    k = 1
    while k < N:
        c = c + _shift_rows(c, k)
        k *= 2
    counts = c[N - 1:N, :]
    rank = jnp.sum(c * onehot, axis=1, keepdims=True) - 1

    tiles = (counts + (TILE - 1)) // TILE
    cuminc = tiles
    k = 1
    while k < E:
        cuminc = cuminc + _shift_lanes(cuminc, k)
        k *= 2
    start = cuminc - tiles
    base = jnp.sum(onehot * (start * TILE), axis=1, keepdims=True)
    dest_ref[...] = base + rank

    rowi = lax.broadcasted_iota(jnp.int32, (LANES, LANES), 0)
    coli = lax.broadcasted_iota(jnp.int32, (LANES, LANES), 1)
    eqmat = (rowi == coli).astype(jnp.float32)
    counts_b = jnp.concatenate([counts.astype(jnp.float32)] * 8, axis=0)
    counts_col = lax.dot_general(eqmat, counts_b,
                                 (((1,), (1,)), ((), ())),
                                 preferred_element_type=jnp.float32)[:, 0:1]
    tiles_col = jnp.floor((counts_col + (TILE - 1)) / TILE)
    cum_col = tiles_col
    k = 1
    while k < E:
        cum_col = cum_col + jnp.concatenate(
            [jnp.zeros((k, 1), jnp.float32), cum_col[:-k, :]], axis=0)
        k *= 2
    start_col = cum_col - tiles_col
    colt = coli.astype(jnp.float32)
    rowe = rowi.astype(jnp.float32)
    mask = (colt >= start_col) & (colt < start_col + tiles_col)
    te = jnp.sum(jnp.where(mask, rowe, 0.0), axis=0, keepdims=True)
    covered = jnp.sum(jnp.where(mask, 1.0, 0.0), axis=0, keepdims=True)
    te = te + (1.0 - covered) * (E - 1)
    lane = coli[0:1, :]
    total_b = jnp.sum(jnp.where(lane == E - 1, cuminc.astype(jnp.float32), 0.0),
                      axis=1, keepdims=True)
    te = jnp.where(lane == MAX_TILES, total_b, te)
    te_ref[...] = te.astype(jnp.int32)


def _router(x_flat, gw_pad):
    return pl.pallas_call(
        _router_body,
        out_shape=(jax.ShapeDtypeStruct((N, 1), jnp.int32),
                   jax.ShapeDtypeStruct((1, LANES), jnp.int32)),
    )(x_flat, gw_pad)


def _ffn_body(te_ref, x_ref, w1_ref, w3_ref, w2_ref, y_ref):
    @pl.when(pl.program_id(0) < te_ref[MAX_TILES])
    def _():
        x = x_ref[...].astype(jnp.bfloat16)
        a = lax.dot_general(x, w1_ref[0].astype(jnp.bfloat16), (((1,), (1,)), ((), ())),
                            preferred_element_type=jnp.float32)
        b = lax.dot_general(x, w3_ref[0].astype(jnp.bfloat16), (((1,), (1,)), ((), ())),
                            preferred_element_type=jnp.float32)
        h = (a * jax.nn.sigmoid(a) * b).astype(jnp.bfloat16)
        y_ref[...] = lax.dot_general(h, w2_ref[0].astype(jnp.bfloat16), (((1,), (1,)), ((), ())),
                                     preferred_element_type=jnp.float32)


def _ffn(x_sorted, w1, w3, w2, te):
    grid_spec = pltpu.PrefetchScalarGridSpec(
        num_scalar_prefetch=1,
        grid=(MAX_TILES,),
        in_specs=[
            pl.BlockSpec((TILE, DIM), lambda t, te: (t, 0)),
            pl.BlockSpec((1, HIDDEN, DIM), lambda t, te: (te[t], 0, 0)),
            pl.BlockSpec((1, HIDDEN, DIM), lambda t, te: (te[t], 0, 0)),
            pl.BlockSpec((1, DIM, HIDDEN), lambda t, te: (te[t], 0, 0)),
        ],
        out_specs=pl.BlockSpec((TILE, DIM), lambda t, te: (t, 0)),
    )
    return pl.pallas_call(
        _ffn_body,
        grid_spec=grid_spec,
        out_shape=jax.ShapeDtypeStruct((PAD_N, DIM), jnp.float32),
    )(te, x_sorted, w1, w3, w2)


@functools.cache
def _vmesh():
    return plsc.VectorSubcoreMesh(core_axis_name="core", subcore_axis_name="subcore")


@functools.cache
def _num_cores():
    return plsc.get_sparse_core_info().num_cores


def _dispatch(x_flat, dest_1d):
    nc = _num_cores()

    @functools.partial(
        pl.kernel,
        out_type=jax.ShapeDtypeStruct((PAD_N, DIM), jnp.float32),
        mesh=_vmesh(),
        scratch_types=[pltpu.VMEM((WINDOW,), jnp.int32),
                       pltpu.VMEM((WINDOW, DIM), jnp.float32),
                       pltpu.SemaphoreType.DMA,
                       pltpu.SemaphoreType.DMA],
    )
    def k(x_hbm, i_hbm, o_hbm, idx_v, rows_v, sem, sem2):
        wid = lax.axis_index("subcore") * nc + lax.axis_index("core")
        base = wid * WINDOW
        cp_i = pltpu.make_async_copy(i_hbm.at[pl.ds(base, WINDOW)], idx_v, sem2)
        cp_x = pltpu.make_async_copy(x_hbm.at[pl.ds(base, WINDOW)], rows_v, sem)
        cp_i.start()
        cp_x.start()
        cp_i.wait()
        cp_x.wait()
        pltpu.async_copy(rows_v, o_hbm.at[idx_v], sem).wait()

    return k(x_flat, dest_1d)


def _combine(y_sorted, dest_1d):
    nc = _num_cores()

    @functools.partial(
        pl.kernel,
        out_type=jax.ShapeDtypeStruct((N, DIM), jnp.float32),
        mesh=_vmesh(),
        scratch_types=[pltpu.VMEM((WINDOW,), jnp.int32),
                       pltpu.VMEM((WINDOW, DIM), jnp.float32),
                       pltpu.SemaphoreType.DMA],
    )
    def k(y_hbm, i_hbm, o_hbm, idx_v, rows_v, sem):
        wid = lax.axis_index("subcore") * nc + lax.axis_index("core")
        base = wid * WINDOW
        pltpu.sync_copy(i_hbm.at[pl.ds(base, WINDOW)], idx_v)
        pltpu.async_copy(y_hbm.at[idx_v], rows_v, sem).wait()
        pltpu.sync_copy(rows_v, o_hbm.at[pl.ds(base, WINDOW)])

    return k(y_sorted, dest_1d)


def kernel(x, gate_w, w1, w2, w3):
    Bv, T, C = x.shape
    x_flat = x.reshape(T, C)
    dest, te = _router(x_flat, gate_w)
    dest_1d = dest.reshape(N)
    te_flat = te.reshape(LANES)[:MAX_TILES + 1]
    x_sorted = _dispatch(x_flat, dest_1d)
    y_sorted = _ffn(x_sorted, w1, w3, w2, te_flat)
    out = _combine(y_sorted, dest_1d)
    return out.reshape(Bv, T, C)


# speedup vs baseline: 1.8769x; 1.0029x over previous
import functools

import jax
import jax.numpy as jnp
from jax import lax
from jax.experimental import pallas as pl
from jax.experimental.pallas import tpu as pltpu
from jax.experimental.pallas import tpu_sc as plsc

DIM = 768
HIDDEN = 2048
E = 8
N = 2048
TILE = 512
MAX_TILES = N // TILE + E - 1
PAD_N = MAX_TILES * TILE
LANES = 128
WINDOW = 64


def _shift_rows(c, k):
    return jnp.concatenate([jnp.zeros((k, c.shape[1]), c.dtype), c[:-k, :]], axis=0)


def _shift_lanes(c, k):
    return jnp.concatenate([jnp.zeros((c.shape[0], k), c.dtype), c[:, :-k]], axis=1)


def _router_body(x_ref, gw_ref, dest_ref, te_ref):
    x = x_ref[...]
    gw = jnp.concatenate(
        [gw_ref[...], jnp.zeros((LANES - E, DIM), jnp.float32)], axis=0)
    logits = lax.dot_general(x, gw, (((1,), (1,)), ((), ())),
                             preferred_element_type=jnp.float32)
    col = lax.broadcasted_iota(jnp.int32, (N, LANES), 1)
    valid = col < E
    lm = jnp.where(valid, logits, -jnp.inf)
    m = jnp.max(lm, axis=1, keepdims=True)
    ex = jnp.exp(lm - m)
    p = ex / jnp.sum(ex, axis=1, keepdims=True)
    pmax = jnp.max(p, axis=1, keepdims=True)
    cand = jnp.where((p == pmax) & valid, col, LANES)
    eid = jnp.min(cand, axis=1, keepdims=True)
    onehot = (col == eid).astype(jnp.int32)

    c = onehot
    k = 1
    while k < N:
        c = c + _shift_rows(c, k)
        k *= 2
    counts = c[N - 1:N, :]
    rank = jnp.sum(c * onehot, axis=1, keepdims=True) - 1

    tiles = (counts + (TILE - 1)) // TILE
    cuminc = tiles
    k = 1
    while k < E:
        cuminc = cuminc + _shift_lanes(cuminc, k)
        k *= 2
    start = cuminc - tiles
    base = jnp.sum(onehot * (start * TILE), axis=1, keepdims=True)
    dest_ref[...] = base + rank

    rowi = lax.broadcasted_iota(jnp.int32, (LANES, LANES), 0)
    coli = lax.broadcasted_iota(jnp.int32, (LANES, LANES), 1)
    eqmat = (rowi == coli).astype(jnp.float32)
    counts_b = jnp.concatenate([counts.astype(jnp.float32)] * 8, axis=0)
    counts_col = lax.dot_general(eqmat, counts_b,
                                 (((1,), (1,)), ((), ())),
                                 preferred_element_type=jnp.float32)[:, 0:1]
    tiles_col = jnp.floor((counts_col + (TILE - 1)) / TILE)
    cum_col = tiles_col
    k = 1
    while k < E:
        cum_col = cum_col + jnp.concatenate(
            [jnp.zeros((k, 1), jnp.float32), cum_col[:-k, :]], axis=0)
        k *= 2
    start_col = cum_col - tiles_col
    colt = coli.astype(jnp.float32)
    rowe = rowi.astype(jnp.float32)
    mask = (colt >= start_col) & (colt < start_col + tiles_col)
    te = jnp.sum(jnp.where(mask, rowe, 0.0), axis=0, keepdims=True)
    covered = jnp.sum(jnp.where(mask, 1.0, 0.0), axis=0, keepdims=True)
    te = te + (1.0 - covered) * (E - 1)
    lane = coli[0:1, :]
    total_b = jnp.sum(jnp.where(lane == E - 1, cuminc.astype(jnp.float32), 0.0),
                      axis=1, keepdims=True)
    te = jnp.where(lane == MAX_TILES, total_b, te)
    te_ref[...] = te.astype(jnp.int32)


def _router(x_flat, gate_w):
    return pl.pallas_call(
        _router_body,
        out_shape=(jax.ShapeDtypeStruct((N, 1), jnp.int32),
                   jax.ShapeDtypeStruct((1, LANES), jnp.int32)),
    )(x_flat, gate_w)


def _ffn_body(te_ref, x_ref, w1_ref, w3_ref, w2_ref, y_ref):
    @pl.when(pl.program_id(0) < te_ref[MAX_TILES])
    def _():
        x = x_ref[...].astype(jnp.bfloat16)
        a = lax.dot_general(x, w1_ref[0].astype(jnp.bfloat16), (((1,), (1,)), ((), ())),
                            preferred_element_type=jnp.float32)
        b = lax.dot_general(x, w3_ref[0].astype(jnp.bfloat16), (((1,), (1,)), ((), ())),
                            preferred_element_type=jnp.float32)
        h = (a * jax.nn.sigmoid(a) * b).astype(jnp.bfloat16)
        y_ref[...] = lax.dot_general(h, w2_ref[0].astype(jnp.bfloat16), (((1,), (1,)), ((), ())),
                                     preferred_element_type=jnp.float32)


def _ffn(x_sorted, w1, w3, w2, te):
    grid_spec = pltpu.PrefetchScalarGridSpec(
        num_scalar_prefetch=1,
        grid=(MAX_TILES,),
        in_specs=[
            pl.BlockSpec((TILE, DIM), lambda t, te: (t, 0)),
            pl.BlockSpec((1, HIDDEN, DIM), lambda t, te: (te[t], 0, 0)),
            pl.BlockSpec((1, HIDDEN, DIM), lambda t, te: (te[t], 0, 0)),
            pl.BlockSpec((1, DIM, HIDDEN), lambda t, te: (te[t], 0, 0)),
        ],
        out_specs=pl.BlockSpec((TILE, DIM), lambda t, te: (t, 0)),
    )
    return pl.pallas_call(
        _ffn_body,
        grid_spec=grid_spec,
        out_shape=jax.ShapeDtypeStruct((PAD_N, DIM), jnp.float32),
    )(te, x_sorted, w1, w3, w2)


@functools.cache
def _vmesh():
    return plsc.VectorSubcoreMesh(core_axis_name="core", subcore_axis_name="subcore")


@functools.cache
def _num_cores():
    return plsc.get_sparse_core_info().num_cores


def _dispatch(x_flat, dest_1d):
    nc = _num_cores()

    @functools.partial(
        pl.kernel,
        out_type=jax.ShapeDtypeStruct((PAD_N, DIM), jnp.float32),
        mesh=_vmesh(),
        scratch_types=[pltpu.VMEM((WINDOW,), jnp.int32),
                       pltpu.VMEM((WINDOW, DIM), jnp.float32),
                       pltpu.SemaphoreType.DMA,
                       pltpu.SemaphoreType.DMA],
    )
    def k(x_hbm, i_hbm, o_hbm, idx_v, rows_v, sem, sem2):
        wid = lax.axis_index("subcore") * nc + lax.axis_index("core")
        base = wid * WINDOW
        cp_i = pltpu.make_async_copy(i_hbm.at[pl.ds(base, WINDOW)], idx_v, sem2)
        cp_x = pltpu.make_async_copy(x_hbm.at[pl.ds(base, WINDOW)], rows_v, sem)
        cp_i.start()
        cp_x.start()
        cp_i.wait()
        cp_x.wait()
        pltpu.async_copy(rows_v, o_hbm.at[idx_v], sem).wait()

    return k(x_flat, dest_1d)


def _combine(y_sorted, dest_1d):
    nc = _num_cores()

    @functools.partial(
        pl.kernel,
        out_type=jax.ShapeDtypeStruct((N, DIM), jnp.float32),
        mesh=_vmesh(),
        scratch_types=[pltpu.VMEM((WINDOW,), jnp.int32),
                       pltpu.VMEM((WINDOW, DIM), jnp.float32),
                       pltpu.SemaphoreType.DMA],
    )
    def k(y_hbm, i_hbm, o_hbm, idx_v, rows_v, sem):
        wid = lax.axis_index("subcore") * nc + lax.axis_index("core")
        base = wid * WINDOW
        pltpu.sync_copy(i_hbm.at[pl.ds(base, WINDOW)], idx_v)
        pltpu.async_copy(y_hbm.at[idx_v], rows_v, sem).wait()
        pltpu.sync_copy(rows_v, o_hbm.at[pl.ds(base, WINDOW)])

    return k(y_sorted, dest_1d)


def kernel(x, gate_w, w1, w2, w3):
    Bv, T, C = x.shape
    x_flat = x.reshape(T, C)
    dest, te = _router(x_flat, gate_w)
    dest_1d = dest.reshape(N)
    te_flat = te.reshape(LANES)[:MAX_TILES + 1]
    x_sorted = _dispatch(x_flat, dest_1d)
    y_sorted = _ffn(x_sorted, w1, w3, w2, te_flat)
    out = _combine(y_sorted, dest_1d)
    return out.reshape(Bv, T, C)
